# baseline (speedup 1.0000x reference)
import functools

import jax
import jax.numpy as jnp
from jax import lax
from jax.experimental import pallas as pl
from jax.experimental.pallas import tpu as pltpu
from jax.experimental.pallas import tpu_sc as plsc

_DELTA = 10.0
_LEAK = 0.01
_NC = 2
_NS = 16
_NW = _NC * _NS
_CHUNK = 128


def _sc_gather(table, idx3):
    nw, ch, lch = idx3.shape
    epw = ch * lch
    c = table.shape[1]
    grp = 8
    mesh = plsc.VectorSubcoreMesh(core_axis_name="c", subcore_axis_name="s")

    @functools.partial(
        pl.kernel,
        out_type=jax.ShapeDtypeStruct((nw, epw, c), jnp.float32),
        mesh=mesh,
        scratch_types=[
            pltpu.VMEM((ch, lch), jnp.int32),
            pltpu.VMEM((epw, c), jnp.float32),
            pltpu.SemaphoreType.DMA,
        ],
        compiler_params=pltpu.CompilerParams(use_tc_tiling_on_sc=False),
    )
    def body(table_hbm, idx_hbm, out_hbm, idx_v, rows_v, sem):
        wid = lax.axis_index("s") * _NC + lax.axis_index("c")
        pltpu.sync_copy(idx_hbm.at[wid], idx_v)

        def group(g, carry):
            handles = []
            for j in range(grp):
                cidx = g * grp + j
                handles.append(
                    pltpu.async_copy(
                        table_hbm.at[idx_v.at[cidx]],
                        rows_v.at[pl.ds(cidx * lch, lch)],
                        sem,
                    )
                )
            for hnd in handles:
                hnd.wait()
            return carry

        lax.fori_loop(0, ch // grp, group, 0)
        pltpu.sync_copy(rows_v, out_hbm.at[wid])

    return body(table, idx3)


def _tc_call(vertex4, center3, hp, w0t, wrp, wlf, e1, wkt, s_red, e2, s2, wc2,
             b0r, bkr, biasr, *, kk, nb, t, c, cout, rank, rows, wimg,
             interpret=False):
    def body(v_ref, c_ref, hp_ref, w0_ref, wrp_ref, wl_ref, e1_ref, wk_ref,
             s_ref, e2_ref, s2_ref, wc_ref, b0_ref, bk_ref, bias_ref, o_ref):
        bf = jnp.bfloat16
        f32 = jnp.float32
        mm = lambda x, y: jnp.dot(x, y, preferred_element_type=f32)
        ib = pl.program_id(0)
        ctr = c_ref[0]
        w0_ = w0_ref[...].astype(bf)
        wrp_ = wrp_ref[...].astype(bf)
        wl_ = wl_ref[...].astype(bf)
        e1_ = e1_ref[...].astype(bf)
        wk_ = wk_ref[...].astype(bf)
        s_ = s_ref[...].astype(bf)
        e2_ = e2_ref[...].astype(bf)
        s2_ = s2_ref[...].astype(bf)
        b0_ = b0_ref[...]
        bk_ = bk_ref[...]
        ctr_pre = mm(ctr.astype(bf), w0_)
        acc = jnp.zeros((t, cout), f32)
        for k in range(kk):
            v = v_ref[k, 0]
            vb = v.astype(bf)
            lab = v - ctr
            pre = mm(vb, w0_) - ctr_pre + b0_
            theta = jnp.where(pre >= 0, pre, _LEAK * pre).astype(bf)
            g = mm(vb, wrp_)
            a_ = mm(theta, wl_)
            th_e = mm(theta, e1_)
            kap = mm(theta, wk_) + bk_
            tmp = mm((th_e * g).astype(bf), s_)
            ssq = jnp.sum(lab * lab, axis=1, keepdims=True)
            gam = jnp.exp(ssq * (-1.0 / _DELTA))
            sv = kap * tmp * gam
            s_e = mm(sv.astype(bf), e2_)
            acc = acc + mm((a_ * s_e).astype(bf), s2_)
        taps = []
        for dy in range(3):
            for dx in range(3):
                blk = hp_ref[pl.ds(ib * rows + dy, rows), pl.ds(dx, wimg), :]
                taps.append(blk.reshape(t, c))
        hcat = jnp.concatenate(taps, axis=1)
        h_l = mm(hcat.astype(bf), wc_ref[...].astype(bf))
        o_ref[0] = acc * (0.5 / kk) + h_l * 0.5 + bias_ref[...]

    full = lambda a: pl.BlockSpec(a.shape, lambda i: (0,) * a.ndim)
    return pl.pallas_call(
        body,
        grid=(nb,),
        in_specs=[
            pl.BlockSpec((kk, 1, t, c), lambda i: (0, i, 0, 0)),
            pl.BlockSpec((1, t, c), lambda i: (i, 0, 0)),
            full(hp), full(w0t), full(wrp), full(wlf), full(e1), full(wkt),
            full(s_red), full(e2), full(s2), full(wc2), full(b0r), full(bkr),
            full(biasr),
        ],
        out_specs=pl.BlockSpec((1, t, cout), lambda i: (i, 0, 0)),
        out_shape=jax.ShapeDtypeStruct((nb, t, cout), jnp.float32),
        interpret=interpret,
    )(vertex4, center3, hp, w0t, wrp, wlf, e1, wkt, s_red, e2, s2, wc2,
      b0r, bkr, biasr)


def _weights(W0, b0, wL, wR, Wk, bk, Wc, bias, c, cout, rank):
    m = (rank * cout) // wL.shape[0]
    cols = jnp.arange(rank * c)
    irow = jnp.arange(c)

    def circfull(w):
        wm = w[:, 0, :]
        return wm[cols[None, :] // m, (irow[:, None] - cols[None, :] % m) % c]

    wlf = circfull(wL)
    wrf = circfull(wR)
    wrp = wrf.reshape(c, c, rank).transpose(1, 0, 2).reshape(c, c * rank)
    e1 = (cols[None, :] // rank == jnp.arange(c)[:, None]).astype(jnp.float32)
    s_red = (cols[:, None] % rank == jnp.arange(rank)[None, :]).astype(jnp.float32)
    e2 = (cols[None, :] % rank == jnp.arange(rank)[:, None]).astype(jnp.float32)
    s2 = (cols[:, None] // rank == jnp.arange(cout)[None, :]).astype(jnp.float32)
    wc2 = Wc.transpose(2, 3, 1, 0).reshape(9 * c, cout)
    return (W0.T, wrp, wlf, e1, Wk.T, s_red, e2, s2, wc2,
            b0.reshape(1, c), bk.reshape(1, rank), bias.reshape(1, cout))


def kernel(h, edge, W0, b0, wL, wR, Wk, bk, Wc, bias):
    b, c, himg, wimg = h.shape
    kk = edge.shape[1]
    n = himg * wimg
    cout = Wc.shape[0]
    rank = Wk.shape[0]
    e_tot = kk * n

    table = h.reshape(c, n).T
    epw = e_tot // _NW
    idx3 = edge.reshape(_NW, epw // _CHUNK, _CHUNK)
    vertex = _sc_gather(table, idx3)

    rows = 16
    t = rows * wimg
    nb = n // t
    vertex4 = vertex.reshape(kk, nb, t, c)
    center3 = table.reshape(nb, t, c)
    hp = jnp.pad(h[0], ((0, 0), (1, 1), (1, 1)), mode="reflect").transpose(1, 2, 0)

    ws = _weights(W0, b0, wL, wR, Wk, bk, Wc, bias, c, cout, rank)
    out_pm = _tc_call(vertex4, center3, hp, *ws, kk=kk, nb=nb, t=t, c=c,
                      cout=cout, rank=rank, rows=rows, wimg=wimg)
    return out_pm.reshape(n, cout).T.reshape(b, cout, himg, wimg)


# speedup vs baseline: 1.1121x; 1.1121x over previous
import functools

import jax
import jax.numpy as jnp
from jax import lax
from jax.experimental import pallas as pl
from jax.experimental.pallas import tpu as pltpu
from jax.experimental.pallas import tpu_sc as plsc

_DELTA = 10.0
_LEAK = 0.01
_NC = 2
_NS = 16
_NW = _NC * _NS
_CHUNK = 128


def _sc_gather(table, idx3):
    nw, ch, lch = idx3.shape
    epw = ch * lch
    c = table.shape[1]
    grp = 8
    mesh = plsc.VectorSubcoreMesh(core_axis_name="c", subcore_axis_name="s")

    @functools.partial(
        pl.kernel,
        out_type=jax.ShapeDtypeStruct((nw, epw, c), jnp.float32),
        mesh=mesh,
        scratch_types=[
            pltpu.VMEM((ch, lch), jnp.int32),
            pltpu.VMEM((epw, c), jnp.float32),
            pltpu.SemaphoreType.DMA,
        ],
        compiler_params=pltpu.CompilerParams(use_tc_tiling_on_sc=False),
    )
    def body(table_hbm, idx_hbm, out_hbm, idx_v, rows_v, sem):
        wid = lax.axis_index("s") * _NC + lax.axis_index("c")
        pltpu.sync_copy(idx_hbm.at[wid], idx_v)

        def group(g, carry):
            handles = []
            for j in range(grp):
                cidx = g * grp + j
                handles.append(
                    pltpu.async_copy(
                        table_hbm.at[idx_v.at[cidx]],
                        rows_v.at[pl.ds(cidx * lch, lch)],
                        sem,
                    )
                )
            for hnd in handles:
                hnd.wait()
            return carry

        lax.fori_loop(0, ch // grp, group, 0)
        pltpu.sync_copy(rows_v, out_hbm.at[wid])

    return body(table, idx3)


def _tc_call(vertex4, center3, hshift, comb0_w, combt_w, s_t, e2_t, s2_t, wc_t,
             b0c, bkc, biasc, *, kk, nb, t, c, cout, rank, interpret=False):
    rc = rank * c

    def body(v_ref, c_ref, hs_ref, c0_ref, ct_ref, s_ref, e2_ref, s2_ref,
             wc_ref, b0_ref, bk_ref, bias_ref, o_ref):
        bf = jnp.bfloat16
        f32 = jnp.float32
        mm = lambda x, y: jnp.dot(x, y, preferred_element_type=f32)
        ctr = c_ref[0]
        ctrb = ctr.astype(bf)
        c0w = c0_ref[...].astype(bf)
        ctw = ct_ref[...].astype(bf)
        s_w = s_ref[...].astype(bf)
        e2w = e2_ref[...].astype(bf)
        s2w = s2_ref[...].astype(bf)
        b0_ = b0_ref[...]
        bk_ = bk_ref[...]
        ctr_pre = mm(c0w[:c], ctrb)
        acc = jnp.zeros((cout, t), f32)
        for k in range(kk):
            vt = v_ref[k, 0]
            vtb = vt.astype(bf)
            comb0 = mm(c0w, vtb)
            pre = comb0[:c] - ctr_pre + b0_
            g = comb0[c:]
            lab = vt - ctr
            theta = jnp.where(pre >= 0, pre, _LEAK * pre).astype(bf)
            combt = mm(ctw, theta)
            a_ = combt[:rc]
            th_e = combt[rc:2 * rc]
            kap = combt[2 * rc:] + bk_
            tmp = mm(s_w, (th_e * g).astype(bf))
            ssq = jnp.sum(lab * lab, axis=0, keepdims=True)
            gam = jnp.exp(ssq * (-1.0 / _DELTA))
            sv = kap * tmp * gam
            s_e = mm(e2w, sv.astype(bf))
            acc = acc + mm(s2w, (a_ * s_e).astype(bf))
        hcat = hs_ref[...].reshape(9 * c, t)
        h_l = mm(wc_ref[...].astype(bf), hcat.astype(bf))
        o_ref[0] = acc * (0.5 / kk) + h_l * 0.5 + bias_ref[...]

    full = lambda a: pl.BlockSpec(a.shape, lambda i: (0,) * a.ndim)
    return pl.pallas_call(
        body,
        grid=(nb,),
        in_specs=[
            pl.BlockSpec((kk, 1, c, t), lambda i: (0, i, 0, 0)),
            pl.BlockSpec((1, c, t), lambda i: (i, 0, 0)),
            pl.BlockSpec((9, 1, c, t), lambda i: (0, i, 0, 0)),
            full(comb0_w), full(combt_w), full(s_t), full(e2_t), full(s2_t),
            full(wc_t), full(b0c), full(bkc), full(biasc),
        ],
        out_specs=pl.BlockSpec((1, cout, t), lambda i: (i, 0, 0)),
        out_shape=jax.ShapeDtypeStruct((nb, cout, t), jnp.float32),
        interpret=interpret,
    )(vertex4, center3, hshift, comb0_w, combt_w, s_t, e2_t, s2_t, wc_t,
      b0c, bkc, biasc)


def _weights(W0, b0, wL, wR, Wk, bk, Wc, bias, c, cout, rank):
    m = (rank * cout) // wL.shape[0]
    cols = jnp.arange(rank * c)
    irow = jnp.arange(c)

    def circfull(w):
        wm = w[:, 0, :]
        return wm[cols[None, :] // m, (irow[:, None] - cols[None, :] % m) % c]

    wlf = circfull(wL)
    wrf = circfull(wR)
    wrp = wrf.reshape(c, c, rank).transpose(1, 0, 2).reshape(c, c * rank)
    e1 = (cols[None, :] // rank == jnp.arange(c)[:, None]).astype(jnp.float32)
    s_red = (cols[:, None] % rank == jnp.arange(rank)[None, :]).astype(jnp.float32)
    e2 = (cols[None, :] % rank == jnp.arange(rank)[:, None]).astype(jnp.float32)
    s2 = (cols[:, None] // rank == jnp.arange(cout)[None, :]).astype(jnp.float32)
    wc2 = Wc.transpose(2, 3, 1, 0).reshape(9 * c, cout)
    comb0_w = jnp.concatenate([W0, wrp.T], axis=0)
    combt_w = jnp.concatenate([wlf.T, e1.T, Wk], axis=0)
    return (comb0_w, combt_w, s_red.T, e2.T, s2.T, wc2.T,
            b0.reshape(c, 1), bk.reshape(rank, 1), bias.reshape(cout, 1))


def kernel(h, edge, W0, b0, wL, wR, Wk, bk, Wc, bias):
    b, c, himg, wimg = h.shape
    kk = edge.shape[1]
    n = himg * wimg
    cout = Wc.shape[0]
    rank = Wk.shape[0]
    e_tot = kk * n

    table = h.reshape(c, n).T
    epw = e_tot // _NW
    idx3 = edge.reshape(_NW, epw // _CHUNK, _CHUNK)
    vertex = _sc_gather(table, idx3)

    rows = 16
    t = rows * wimg
    nb = n // t
    vertex4 = vertex.reshape(kk, nb, t, c).transpose(0, 1, 3, 2)
    center3 = h.reshape(c, nb, t).transpose(1, 0, 2)
    hp = jnp.pad(h[0], ((0, 0), (1, 1), (1, 1)), mode="reflect")
    taps = [hp[:, dy:dy + himg, dx:dx + wimg].reshape(c, nb, t)
            for dy in range(3) for dx in range(3)]
    hshift = jnp.stack(taps, axis=0).transpose(0, 2, 1, 3)

    ws = _weights(W0, b0, wL, wR, Wk, bk, Wc, bias, c, cout, rank)
    out_cm = _tc_call(vertex4, center3, hshift, *ws, kk=kk, nb=nb, t=t, c=c,
                      cout=cout, rank=rank)
    return out_cm.transpose(1, 0, 2).reshape(b, cout, himg, wimg)


# speedup vs baseline: 1.6704x; 1.5021x over previous
import functools

import jax
import jax.numpy as jnp
from jax import lax
from jax.experimental import pallas as pl
from jax.experimental.pallas import tpu as pltpu
from jax.experimental.pallas import tpu_sc as plsc

_DELTA = 10.0
_LEAK = 0.01
_NC = 2
_NS = 16
_NW = _NC * _NS
_CHUNK = 128


def _sc_gather(table, idx3, kk, n):
    nw, ch, lch = idx3.shape
    epw = ch * lch
    c = table.shape[1]
    wpk = n // epw
    grp = 8
    mesh = plsc.VectorSubcoreMesh(core_axis_name="c", subcore_axis_name="s")

    @functools.partial(
        pl.kernel,
        out_type=jax.ShapeDtypeStruct((kk, c, n), jnp.float32),
        mesh=mesh,
        scratch_types=[
            pltpu.VMEM((ch, lch), jnp.int32),
            pltpu.VMEM((grp * lch, c), jnp.float32),
            pltpu.VMEM((c, epw), jnp.float32),
            pltpu.SemaphoreType.DMA,
        ],
        compiler_params=pltpu.CompilerParams(use_tc_tiling_on_sc=False,
                                             needs_layout_passes=False),
    )
    def body(table_hbm, idx_hbm, out_hbm, idx_v, rows_v, trans_v, sem):
        wid = lax.axis_index("s") * _NC + lax.axis_index("c")
        k0 = wid // wpk
        n0 = (wid % wpk) * epw
        pltpu.sync_copy(idx_hbm.at[wid], idx_v)
        lanes = lax.iota(jnp.int32, 16)

        def group(g, carry):
            handles = []
            for j in range(grp):
                cidx = g * grp + j
                handles.append(
                    pltpu.async_copy(
                        table_hbm.at[idx_v.at[cidx]],
                        rows_v.at[pl.ds(j * lch, lch)],
                        sem,
                    )
                )
            for hnd in handles:
                hnd.wait()
            base = g * (grp * lch)

            def pix(p8, carry2):
                for q in range(8):
                    p = p8 * 8 + q
                    plsc.store_scatter(
                        trans_v,
                        [lanes, jnp.full((16,), base + p, jnp.int32)],
                        rows_v[p],
                    )
                return carry2

            lax.fori_loop(0, (grp * lch) // 8, pix, 0)
            return carry

        lax.fori_loop(0, ch // grp, group, 0)
        pltpu.sync_copy(trans_v, out_hbm.at[k0, :, pl.ds(n0, epw)])

    return body(table, idx3)


def _tc_call(vertex4, center3, hshift, comb0_w, combt_w, s_t, e2_t, s2_t, wc_t,
             b0c, bkc, biasc, *, kk, nb, t, c, cout, rank, interpret=False):
    rc = rank * c

    def body(v_ref, c_ref, hs_ref, c0_ref, ct_ref, s_ref, e2_ref, s2_ref,
             wc_ref, b0_ref, bk_ref, bias_ref, o_ref):
        bf = jnp.bfloat16
        f32 = jnp.float32
        mm = lambda x, y: jnp.dot(x, y, preferred_element_type=f32)
        ctr = c_ref[...]
        ctrb = ctr.astype(bf)
        c0w = c0_ref[...].astype(bf)
        ctw = ct_ref[...].astype(bf)
        s_w = s_ref[...].astype(bf)
        e2w = e2_ref[...].astype(bf)
        s2w = s2_ref[...].astype(bf)
        b0_ = b0_ref[...]
        bk_ = bk_ref[...]
        ctr_pre = mm(c0w[:c], ctrb)
        acc = jnp.zeros((cout, t), f32)
        for k in range(kk):
            vt = v_ref[k]
            vtb = vt.astype(bf)
            comb0 = mm(c0w, vtb)
            pre = comb0[:c] - ctr_pre + b0_
            g = comb0[c:]
            lab = vt - ctr
            theta = jnp.where(pre >= 0, pre, _LEAK * pre).astype(bf)
            combt = mm(ctw, theta)
            a_ = combt[:rc]
            th_e = combt[rc:2 * rc]
            kap = combt[2 * rc:] + bk_
            tmp = mm(s_w, (th_e * g).astype(bf))
            ssq = jnp.sum(lab * lab, axis=0, keepdims=True)
            gam = jnp.exp(ssq * (-1.0 / _DELTA))
            sv = kap * tmp * gam
            s_e = mm(e2w, sv.astype(bf))
            acc = acc + mm(s2w, (a_ * s_e).astype(bf))
        hcat = hs_ref[...].reshape(9 * c, t)
        h_l = mm(wc_ref[...].astype(bf), hcat.astype(bf))
        o_ref[...] = acc * (0.5 / kk) + h_l * 0.5 + bias_ref[...]

    full = lambda a: pl.BlockSpec(a.shape, lambda i: (0,) * a.ndim)
    n_all = nb * t
    return pl.pallas_call(
        body,
        grid=(nb,),
        in_specs=[
            pl.BlockSpec((kk, c, t), lambda i: (0, 0, i)),
            pl.BlockSpec((c, t), lambda i: (0, i)),
            pl.BlockSpec((9, c, t), lambda i: (0, 0, i)),
            full(comb0_w), full(combt_w), full(s_t), full(e2_t), full(s2_t),
            full(wc_t), full(b0c), full(bkc), full(biasc),
        ],
        out_specs=pl.BlockSpec((cout, t), lambda i: (0, i)),
        out_shape=jax.ShapeDtypeStruct((cout, n_all), jnp.float32),
        interpret=interpret,
    )(vertex4, center3, hshift, comb0_w, combt_w, s_t, e2_t, s2_t, wc_t,
      b0c, bkc, biasc)


def _weights(W0, b0, wL, wR, Wk, bk, Wc, bias, c, cout, rank):
    m = (rank * cout) // wL.shape[0]
    cols = jnp.arange(rank * c)

    def circfull(w):
        wm = w[:, 0, :]
        rolls = jnp.stack([jnp.roll(wm, tt, axis=1) for tt in range(m)],
                          axis=1)
        return rolls.reshape(rank * c, c).T

    wlf = circfull(wL)
    wrf = circfull(wR)
    wrp = wrf.reshape(c, c, rank).transpose(1, 0, 2).reshape(c, c * rank)
    e1 = (cols[None, :] // rank == jnp.arange(c)[:, None]).astype(jnp.float32)
    s_red = (cols[:, None] % rank == jnp.arange(rank)[None, :]).astype(jnp.float32)
    e2 = (cols[None, :] % rank == jnp.arange(rank)[:, None]).astype(jnp.float32)
    s2 = (cols[:, None] // rank == jnp.arange(cout)[None, :]).astype(jnp.float32)
    wc2 = Wc.transpose(2, 3, 1, 0).reshape(9 * c, cout)
    comb0_w = jnp.concatenate([W0, wrp.T], axis=0)
    combt_w = jnp.concatenate([wlf.T, e1.T, Wk], axis=0)
    return (comb0_w, combt_w, s_red.T, e2.T, s2.T, wc2.T,
            b0.reshape(c, 1), bk.reshape(rank, 1), bias.reshape(cout, 1))


def kernel(h, edge, W0, b0, wL, wR, Wk, bk, Wc, bias):
    b, c, himg, wimg = h.shape
    kk = edge.shape[1]
    n = himg * wimg
    cout = Wc.shape[0]
    rank = Wk.shape[0]
    e_tot = kk * n

    h_cm = h.reshape(c, n)
    table = h_cm.T
    epw = e_tot // _NW
    idx3 = edge.reshape(_NW, epw // _CHUNK, _CHUNK)
    vertex = _sc_gather(table, idx3, kk, n)

    rows = 16
    t = rows * wimg
    nb = n // t
    hp = jnp.pad(h[0], ((0, 0), (1, 1), (1, 1)), mode="reflect")
    taps = [hp[:, dy:dy + himg, dx:dx + wimg].reshape(c, n)
            for dy in range(3) for dx in range(3)]
    hshift = jnp.stack(taps, axis=0)

    ws = _weights(W0, b0, wL, wR, Wk, bk, Wc, bias, c, cout, rank)
    out_cm = _tc_call(vertex, h_cm, hshift, *ws, kk=kk, nb=nb, t=t, c=c,
                      cout=cout, rank=rank)
    return out_cm.reshape(b, cout, himg, wimg)


# speedup vs baseline: 1.9015x; 1.1384x over previous
import functools

import jax
import jax.numpy as jnp
from jax import lax
from jax.experimental import pallas as pl
from jax.experimental.pallas import tpu as pltpu
from jax.experimental.pallas import tpu_sc as plsc

_DELTA = 10.0
_LEAK = 0.01
_NC = 2
_NS = 16
_NW = _NC * _NS
_CHUNK = 128


def _sc_gather(table, idx3, kk, n):
    nw, ch, lch = idx3.shape
    epw = ch * lch
    c = table.shape[1]
    wpk = n // epw
    grp = 8
    mesh = plsc.VectorSubcoreMesh(core_axis_name="c", subcore_axis_name="s")

    @functools.partial(
        pl.kernel,
        out_type=jax.ShapeDtypeStruct((kk, c, n), jnp.float32),
        mesh=mesh,
        scratch_types=[
            pltpu.VMEM((ch, lch), jnp.int32),
            pltpu.VMEM((grp * lch, c), jnp.float32),
            pltpu.VMEM((c, epw), jnp.float32),
            pltpu.SemaphoreType.DMA,
        ],
        compiler_params=pltpu.CompilerParams(use_tc_tiling_on_sc=False,
                                             needs_layout_passes=False),
    )
    def body(table_hbm, idx_hbm, out_hbm, idx_v, rows_v, trans_v, sem):
        wid = lax.axis_index("s") * _NC + lax.axis_index("c")
        k0 = wid // wpk
        n0 = (wid % wpk) * epw
        pltpu.sync_copy(idx_hbm.at[wid], idx_v)
        lanes = lax.iota(jnp.int32, 16)

        def group(g, carry):
            handles = []
            for j in range(grp):
                cidx = g * grp + j
                handles.append(
                    pltpu.async_copy(
                        table_hbm.at[idx_v.at[cidx]],
                        rows_v.at[pl.ds(j * lch, lch)],
                        sem,
                    )
                )
            for hnd in handles:
                hnd.wait()
            base = g * (grp * lch)

            def pix(p8, carry2):
                for q in range(8):
                    p = p8 * 8 + q
                    plsc.store_scatter(
                        trans_v,
                        [lanes, jnp.full((16,), base + p, jnp.int32)],
                        rows_v[p],
                    )
                return carry2

            lax.fori_loop(0, (grp * lch) // 8, pix, 0)
            return carry

        lax.fori_loop(0, ch // grp, group, 0)
        pltpu.sync_copy(trans_v, out_hbm.at[k0, :, pl.ds(n0, epw)])

    return body(table, idx3)


def _tc_call(vertex4, center3, hshift, comb0_w, combt_w, s_t, s2_t, wc_t,
             b0c, bkc, biasc, *, kk, nb, t, c, cout, rank, interpret=False):
    rc = rank * c

    def body(v_ref, c_ref, hs_ref, c0_ref, ct_ref, s_ref, s2_ref,
             wc_ref, b0_ref, bk_ref, bias_ref, o_ref):
        bf = jnp.bfloat16
        f32 = jnp.float32
        mm = lambda x, y: jnp.dot(x, y, preferred_element_type=f32)
        ctr = c_ref[...]
        ctrb = ctr.astype(bf)
        c0w = c0_ref[...].astype(bf)
        ctw = ct_ref[...].astype(bf)
        s_w = s_ref[...].astype(bf)
        s2w = s2_ref[...].astype(bf)
        b0_ = b0_ref[...]
        bk_ = bk_ref[...]
        ctr_pre = mm(c0w[:c], ctrb)
        acc = jnp.zeros((cout, t), f32)
        for k in range(kk):
            vt = v_ref[k]
            vtb = vt.astype(bf)
            comb0 = mm(c0w, vtb)
            pre = comb0[:c] - ctr_pre + b0_
            g = comb0[c:]
            lab = vt - ctr
            theta = jnp.where(pre >= 0, pre, _LEAK * pre).astype(bf)
            combt = mm(ctw, theta)
            a_ = combt[:rc]
            kap = combt[rc:] + bk_
            th_e = jnp.repeat(theta.astype(f32), rank, axis=0)
            tmp = mm(s_w, (th_e * g).astype(bf))
            ssq = jnp.sum(lab * lab, axis=0, keepdims=True)
            gam = jnp.exp(ssq * (-1.0 / _DELTA))
            sv = kap * tmp * gam
            s_e = jnp.tile(sv, (cout, 1))
            acc = acc + mm(s2w, (a_ * s_e).astype(bf))
        hcat = hs_ref[...].reshape(9 * c, t)
        h_l = mm(wc_ref[...].astype(bf), hcat.astype(bf))
        o_ref[...] = acc * (0.5 / kk) + h_l * 0.5 + bias_ref[...]

    full = lambda a: pl.BlockSpec(a.shape, lambda i: (0,) * a.ndim)
    n_all = nb * t
    return pl.pallas_call(
        body,
        grid=(nb,),
        in_specs=[
            pl.BlockSpec((kk, c, t), lambda i: (0, 0, i)),
            pl.BlockSpec((c, t), lambda i: (0, i)),
            pl.BlockSpec((9, c, t), lambda i: (0, 0, i)),
            full(comb0_w), full(combt_w), full(s_t), full(s2_t),
            full(wc_t), full(b0c), full(bkc), full(biasc),
        ],
        out_specs=pl.BlockSpec((cout, t), lambda i: (0, i)),
        out_shape=jax.ShapeDtypeStruct((cout, n_all), jnp.float32),
        interpret=interpret,
    )(vertex4, center3, hshift, comb0_w, combt_w, s_t, s2_t, wc_t,
      b0c, bkc, biasc)


def _weights(W0, b0, wL, wR, Wk, bk, Wc, bias, c, cout, rank):
    m = (rank * cout) // wL.shape[0]
    cols = jnp.arange(rank * c)

    def circfull(w):
        wm = w[:, 0, :]
        rolls = jnp.stack([jnp.roll(wm, tt, axis=1) for tt in range(m)],
                          axis=1)
        return rolls.reshape(rank * c, c).T

    wlf = circfull(wL)
    wrf = circfull(wR)
    wrp = wrf.reshape(c, c, rank).transpose(1, 0, 2).reshape(c, c * rank)
    s_red = (cols[:, None] % rank == jnp.arange(rank)[None, :]).astype(jnp.float32)
    s2 = (cols[:, None] // rank == jnp.arange(cout)[None, :]).astype(jnp.float32)
    wc2 = Wc.transpose(2, 3, 1, 0).reshape(9 * c, cout)
    comb0_w = jnp.concatenate([W0, wrp.T], axis=0)
    combt_w = jnp.concatenate([wlf.T, Wk], axis=0)
    return (comb0_w, combt_w, s_red.T, s2.T, wc2.T,
            b0.reshape(c, 1), bk.reshape(rank, 1), bias.reshape(cout, 1))


def kernel(h, edge, W0, b0, wL, wR, Wk, bk, Wc, bias):
    b, c, himg, wimg = h.shape
    kk = edge.shape[1]
    n = himg * wimg
    cout = Wc.shape[0]
    rank = Wk.shape[0]
    e_tot = kk * n

    h_cm = h.reshape(c, n)
    table = h_cm.T
    epw = e_tot // _NW
    idx3 = edge.reshape(_NW, epw // _CHUNK, _CHUNK)
    vertex = _sc_gather(table, idx3, kk, n)

    rows = 16
    t = rows * wimg
    nb = n // t
    hp = jnp.pad(h[0], ((0, 0), (1, 1), (1, 1)), mode="reflect")
    taps = [hp[:, dy:dy + himg, dx:dx + wimg].reshape(c, n)
            for dy in range(3) for dx in range(3)]
    hshift = jnp.stack(taps, axis=0)

    ws = _weights(W0, b0, wL, wR, Wk, bk, Wc, bias, c, cout, rank)
    out_cm = _tc_call(vertex, h_cm, hshift, *ws, kk=kk, nb=nb, t=t, c=c,
                      cout=cout, rank=rank)
    return out_cm.reshape(b, cout, himg, wimg)
